# baseline (speedup 1.0000x reference)
import jax
import jax.numpy as jnp
from jax.experimental import pallas as pl
from jax.experimental.pallas import tpu as pltpu

S, D = 2048, 1024
H, HD = 16, 64
E, K, HID = 8, 2, 2048

HIGHEST = jax.lax.Precision.HIGHEST


def _ln_f32(x, w, b):
    m = jnp.mean(x, axis=-1, keepdims=True)
    v = jnp.mean((x - m) ** 2, axis=-1, keepdims=True)
    return (x - m) / jnp.sqrt(v + 1e-5) * w + b


def _ln_qkv_body(x_ref, w_ref, lnw_ref, lnb_ref, b_ref, out_ref):
    h = _ln_f32(x_ref[...], lnw_ref[...], lnb_ref[...])
    out_ref[...] = (
        jnp.dot(h, w_ref[...], precision=HIGHEST,
                preferred_element_type=jnp.float32)
        + b_ref[...]
    )


def _ln_qkv(x, qkv_W, qkv_b, ln1_w, ln1_b):
    SB, NB = 256, 1024
    return pl.pallas_call(
        _ln_qkv_body,
        grid=(3, S // SB),
        in_specs=[
            pl.BlockSpec((SB, D), lambda nb, sb: (sb, 0)),
            pl.BlockSpec((D, NB), lambda nb, sb: (0, nb)),
            pl.BlockSpec((D,), lambda nb, sb: (0,)),
            pl.BlockSpec((D,), lambda nb, sb: (0,)),
            pl.BlockSpec((NB,), lambda nb, sb: (nb,)),
        ],
        out_specs=pl.BlockSpec((SB, NB), lambda nb, sb: (sb, nb)),
        out_shape=jax.ShapeDtypeStruct((S, 3 * D), jnp.float32),
    )(x, qkv_W, ln1_w, ln1_b, qkv_b)


def _attn_body(q_ref, k_ref, v_ref, out_ref):
    q = q_ref[0]
    k = k_ref[0]
    s = jax.lax.dot_general(
        q, k, (((1,), (1,)), ((), ())), precision=HIGHEST,
        preferred_element_type=jnp.float32) * (HD ** -0.5)
    m = jnp.max(s, axis=-1, keepdims=True)
    p = jnp.exp(s - m)
    p = p / jnp.sum(p, axis=-1, keepdims=True)
    out_ref[0] = jnp.dot(p, v_ref[0], precision=HIGHEST,
                         preferred_element_type=jnp.float32)


def _attention(qkv3):
    QB = 1024
    return pl.pallas_call(
        _attn_body,
        grid=(H, S // QB),
        in_specs=[
            pl.BlockSpec((1, QB, HD), lambda h, qb: (h, qb, 0)),
            pl.BlockSpec((1, S, HD), lambda h, qb: (H + h, 0, 0)),
            pl.BlockSpec((1, S, HD), lambda h, qb: (2 * H + h, 0, 0)),
        ],
        out_specs=pl.BlockSpec((1, QB, HD), lambda h, qb: (h, qb, 0)),
        out_shape=jax.ShapeDtypeStruct((H, S, HD), jnp.float32),
    )(qkv3, qkv3, qkv3)


def _proj_router_body(ao_ref, x_ref, w_ref, b_ref, ln2w_ref, ln2b_ref,
                      gw_ref, gb_ref, x2_ref, h2_ref, wfull_ref):
    proj = jnp.dot(ao_ref[...], w_ref[...], precision=HIGHEST,
                   preferred_element_type=jnp.float32) + b_ref[...]
    x2 = x_ref[...] + proj
    x2_ref[...] = x2
    h2 = _ln_f32(x2, ln2w_ref[...], ln2b_ref[...])
    h2_ref[...] = h2.astype(jnp.bfloat16)
    g = jnp.dot(h2, gw_ref[...], precision=HIGHEST,
                preferred_element_type=jnp.float32) + gb_ref[...]
    iota = jax.lax.broadcasted_iota(jnp.int32, g.shape, 1)
    m1 = jnp.max(g, axis=-1, keepdims=True)
    i1 = jnp.min(jnp.where(g == m1, iota, E), axis=-1, keepdims=True)
    g2 = jnp.where(iota == i1, -jnp.inf, g)
    m2 = jnp.max(g2, axis=-1, keepdims=True)
    i2 = jnp.min(jnp.where(g2 == m2, iota, E), axis=-1, keepdims=True)
    p1 = 1.0 / (1.0 + jnp.exp(m2 - m1))
    p2 = 1.0 / (1.0 + jnp.exp(m1 - m2))
    wfull_ref[...] = jnp.where(iota == i1, p1, 0.0) + jnp.where(iota == i2, p2, 0.0)


def _proj_router(ao, x, attn_out_W, attn_out_b, ln2_w, ln2_b, gate_W, gate_b):
    SB = 256
    return pl.pallas_call(
        _proj_router_body,
        grid=(S // SB,),
        in_specs=[
            pl.BlockSpec((SB, D), lambda sb: (sb, 0)),
            pl.BlockSpec((SB, D), lambda sb: (sb, 0)),
            pl.BlockSpec((D, D), lambda sb: (0, 0)),
            pl.BlockSpec((D,), lambda sb: (0,)),
            pl.BlockSpec((D,), lambda sb: (0,)),
            pl.BlockSpec((D,), lambda sb: (0,)),
            pl.BlockSpec((D, E), lambda sb: (0, 0)),
            pl.BlockSpec((E,), lambda sb: (0,)),
        ],
        out_specs=[
            pl.BlockSpec((SB, D), lambda sb: (sb, 0)),
            pl.BlockSpec((SB, D), lambda sb: (sb, 0)),
            pl.BlockSpec((SB, E), lambda sb: (sb, 0)),
        ],
        out_shape=[
            jax.ShapeDtypeStruct((S, D), jnp.float32),
            jax.ShapeDtypeStruct((S, D), jnp.bfloat16),
            jax.ShapeDtypeStruct((S, E), jnp.float32),
        ],
    )(ao, x, attn_out_W, attn_out_b, ln2_w, ln2_b, gate_W, gate_b)


def _gelu_exact(u):
    return u * 0.5 * (1.0 + jax.lax.erf(u * (2.0 ** -0.5)))


def _moe_body(h2_ref, wfull_ref, w1_ref, b1_ref, w2_ref, b2_ref, out_ref):
    e = pl.program_id(1)
    u = jnp.dot(h2_ref[...], w1_ref[0], preferred_element_type=jnp.float32)
    u = u + b1_ref[0]
    hid = _gelu_exact(u).astype(jnp.bfloat16)
    y = jnp.dot(hid, w2_ref[0], preferred_element_type=jnp.float32) + b2_ref[0]
    iota = jax.lax.broadcasted_iota(jnp.int32, (1, E), 1)
    w = jnp.sum(jnp.where(iota == e, wfull_ref[...], 0.0), axis=-1,
                keepdims=True)
    contrib = w * y

    @pl.when(e == 0)
    def _init():
        out_ref[...] = contrib

    @pl.when(e > 0)
    def _acc():
        out_ref[...] += contrib


def _moe_dense(h2b, w_full, exp_W1b, exp_b1, exp_W2b, exp_b2):
    SB = 1024
    return pl.pallas_call(
        _moe_body,
        grid=(S // SB, E),
        in_specs=[
            pl.BlockSpec((SB, D), lambda sb, e: (sb, 0)),
            pl.BlockSpec((SB, E), lambda sb, e: (sb, 0)),
            pl.BlockSpec((1, D, HID), lambda sb, e: (e, 0, 0)),
            pl.BlockSpec((1, 1, HID), lambda sb, e: (e, 0, 0)),
            pl.BlockSpec((1, HID, D), lambda sb, e: (e, 0, 0)),
            pl.BlockSpec((1, 1, D), lambda sb, e: (e, 0, 0)),
        ],
        out_specs=pl.BlockSpec((SB, D), lambda sb, e: (sb, 0)),
        out_shape=jax.ShapeDtypeStruct((S, D), jnp.float32),
        compiler_params=pltpu.CompilerParams(
            dimension_semantics=("arbitrary", "arbitrary")),
    )(h2b, w_full, exp_W1b, exp_b1, exp_W2b, exp_b2)


def _final_body(x2_ref, moe_ref, w_ref, b_ref, out_ref):
    z = x2_ref[...] + moe_ref[...]
    m = jnp.sum(z, axis=0, keepdims=True) * (1.0 / S)
    out_ref[...] = jnp.dot(m, w_ref[...], precision=HIGHEST,
                           preferred_element_type=jnp.float32) + b_ref[...]


def _final(x2, moe, fc_W, fc_b):
    return pl.pallas_call(
        _final_body,
        grid=(1,),
        in_specs=[
            pl.BlockSpec((S, D), lambda i: (0, 0)),
            pl.BlockSpec((S, D), lambda i: (0, 0)),
            pl.BlockSpec((D, D), lambda i: (0, 0)),
            pl.BlockSpec((D,), lambda i: (0,)),
        ],
        out_specs=pl.BlockSpec((1, D), lambda i: (0, 0)),
        out_shape=jax.ShapeDtypeStruct((1, D), jnp.float32),
    )(x2, moe, fc_W, fc_b)


def kernel(x, qkv_W, qkv_b, attn_out_W, attn_out_b, gate_W, gate_b,
           exp_W1, exp_b1, exp_W2, exp_b2, ln1_w, ln1_b, ln2_w, ln2_b,
           fc_W, fc_b):
    xs = x.reshape(S, D)
    qkv = _ln_qkv(xs, qkv_W, qkv_b, ln1_w, ln1_b)
    qkv3 = qkv.reshape(S, 3 * H, HD).transpose(1, 0, 2)
    aoh = _attention(qkv3)
    ao = aoh.transpose(1, 0, 2).reshape(S, D)
    x2, h2b, w_full = _proj_router(ao, xs, attn_out_W, attn_out_b,
                                   ln2_w, ln2_b, gate_W, gate_b)
    moe = _moe_dense(h2b, w_full,
                     exp_W1.astype(jnp.bfloat16), exp_b1.reshape(E, 1, HID),
                     exp_W2.astype(jnp.bfloat16), exp_b2.reshape(E, 1, D))
    return _final(x2, moe, fc_W, fc_b)


# speedup vs baseline: 2.0067x; 2.0067x over previous
import jax
import jax.numpy as jnp
from jax.experimental import pallas as pl
from jax.experimental.pallas import tpu as pltpu

S, D = 2048, 1024
H, HD = 16, 64
E, K, HID = 8, 2, 2048

HIGHEST = jax.lax.Precision.HIGHEST
BF = jnp.bfloat16
F32 = jnp.float32


def _split(x):
    hi = x.astype(BF)
    lo = (x - hi.astype(F32)).astype(BF)
    return hi, lo


def _split_w(w):
    hi = w.astype(BF)
    lo = (w - hi.astype(F32)).astype(BF)
    return hi, lo


def _mm3(a_hi, a_lo, b_hi, b_lo, dims):
    dn = (dims, ((), ()))
    t = jax.lax.dot_general(a_hi, b_lo, dn, preferred_element_type=F32)
    t = t + jax.lax.dot_general(a_lo, b_hi, dn, preferred_element_type=F32)
    t = t + jax.lax.dot_general(a_hi, b_hi, dn, preferred_element_type=F32)
    return t


def _ln_f32(x, w, b):
    m = jnp.mean(x, axis=-1, keepdims=True)
    v = jnp.mean((x - m) ** 2, axis=-1, keepdims=True)
    return (x - m) / jnp.sqrt(v + 1e-5) * w + b


def _ln_qkv_body(x_ref, whi_ref, wlo_ref, lnw_ref, lnb_ref, b_ref,
                 hi_ref, lo_ref):
    h = _ln_f32(x_ref[...], lnw_ref[...], lnb_ref[...])
    h_hi, h_lo = _split(h)
    out = _mm3(h_hi, h_lo, whi_ref[...], wlo_ref[...], ((1,), (0,)))
    out = out + b_ref[...]
    o_hi, o_lo = _split(out)
    hi_ref[...] = o_hi
    lo_ref[...] = o_lo


def _ln_qkv(x, qkv_W, qkv_b, ln1_w, ln1_b):
    SB, NB = 256, 1024
    w_hi, w_lo = _split_w(qkv_W)
    return pl.pallas_call(
        _ln_qkv_body,
        grid=(3, S // SB),
        in_specs=[
            pl.BlockSpec((SB, D), lambda nb, sb: (sb, 0)),
            pl.BlockSpec((D, NB), lambda nb, sb: (0, nb)),
            pl.BlockSpec((D, NB), lambda nb, sb: (0, nb)),
            pl.BlockSpec((D,), lambda nb, sb: (0,)),
            pl.BlockSpec((D,), lambda nb, sb: (0,)),
            pl.BlockSpec((NB,), lambda nb, sb: (nb,)),
        ],
        out_specs=[
            pl.BlockSpec((SB, NB), lambda nb, sb: (sb, nb)),
            pl.BlockSpec((SB, NB), lambda nb, sb: (sb, nb)),
        ],
        out_shape=[
            jax.ShapeDtypeStruct((S, 3 * D), BF),
            jax.ShapeDtypeStruct((S, 3 * D), BF),
        ],
    )(x, w_hi, w_lo, ln1_w, ln1_b, qkv_b)


def _attn_body(qhi_ref, qlo_ref, khi_ref, klo_ref, vhi_ref, vlo_ref,
               out_ref):
    pieces = []
    for h in range(H):
        sl = slice(h * HD, (h + 1) * HD)
        q_hi = qhi_ref[:, sl]
        q_lo = qlo_ref[:, sl]
        k_hi = khi_ref[:, sl]
        k_lo = klo_ref[:, sl]
        v_hi = vhi_ref[:, sl]
        v_lo = vlo_ref[:, sl]
        sT = _mm3(k_hi, k_lo, q_hi, q_lo, ((1,), (1,))) * (HD ** -0.5)
        m = jnp.max(sT, axis=0, keepdims=True)
        p = jnp.exp(sT - m)
        rec = 1.0 / jnp.sum(p, axis=0, keepdims=True)
        p_hi, p_lo = _split(p)
        aoT = _mm3(v_hi, v_lo, p_hi, p_lo, ((0,), (0,)))
        pieces.append(aoT * rec)
    out_ref[...] = jnp.concatenate(pieces, axis=0)


def _attention(qkv_hi, qkv_lo):
    QB = 512
    return pl.pallas_call(
        _attn_body,
        grid=(S // QB,),
        in_specs=[
            pl.BlockSpec((QB, D), lambda qb: (qb, 0)),
            pl.BlockSpec((QB, D), lambda qb: (qb, 0)),
            pl.BlockSpec((S, D), lambda qb: (0, 1)),
            pl.BlockSpec((S, D), lambda qb: (0, 1)),
            pl.BlockSpec((S, D), lambda qb: (0, 2)),
            pl.BlockSpec((S, D), lambda qb: (0, 2)),
        ],
        out_specs=pl.BlockSpec((D, QB), lambda qb: (0, qb)),
        out_shape=jax.ShapeDtypeStruct((D, S), F32),
    )(qkv_hi, qkv_lo, qkv_hi, qkv_lo, qkv_hi, qkv_lo)


def _proj_router_body(aoT_ref, x_ref, whi_ref, wlo_ref, b_ref,
                      ln2w_ref, ln2b_ref, gw_ref, gb_ref,
                      x2_ref, h2_ref, wfull_ref):
    ao = aoT_ref[...].T
    a_hi, a_lo = _split(ao)
    proj = _mm3(a_hi, a_lo, whi_ref[...], wlo_ref[...], ((1,), (0,)))
    x2 = x_ref[...] + proj + b_ref[...]
    x2_ref[...] = x2
    h2 = _ln_f32(x2, ln2w_ref[...], ln2b_ref[...])
    h2_ref[...] = h2.astype(BF)
    g = jnp.dot(h2, gw_ref[...], precision=HIGHEST,
                preferred_element_type=F32) + gb_ref[...]
    iota = jax.lax.broadcasted_iota(jnp.int32, g.shape, 1)
    m1 = jnp.max(g, axis=-1, keepdims=True)
    i1 = jnp.min(jnp.where(g == m1, iota, E), axis=-1, keepdims=True)
    g2 = jnp.where(iota == i1, -jnp.inf, g)
    m2 = jnp.max(g2, axis=-1, keepdims=True)
    i2 = jnp.min(jnp.where(g2 == m2, iota, E), axis=-1, keepdims=True)
    p1 = 1.0 / (1.0 + jnp.exp(m2 - m1))
    p2 = 1.0 / (1.0 + jnp.exp(m1 - m2))
    wfull_ref[...] = jnp.where(iota == i1, p1, 0.0) + jnp.where(iota == i2, p2, 0.0)


def _proj_router(aoT, x, attn_out_W, attn_out_b, ln2_w, ln2_b, gate_W, gate_b):
    SB = 256
    w_hi, w_lo = _split_w(attn_out_W)
    return pl.pallas_call(
        _proj_router_body,
        grid=(S // SB,),
        in_specs=[
            pl.BlockSpec((D, SB), lambda sb: (0, sb)),
            pl.BlockSpec((SB, D), lambda sb: (sb, 0)),
            pl.BlockSpec((D, D), lambda sb: (0, 0)),
            pl.BlockSpec((D, D), lambda sb: (0, 0)),
            pl.BlockSpec((D,), lambda sb: (0,)),
            pl.BlockSpec((D,), lambda sb: (0,)),
            pl.BlockSpec((D,), lambda sb: (0,)),
            pl.BlockSpec((D, E), lambda sb: (0, 0)),
            pl.BlockSpec((E,), lambda sb: (0,)),
        ],
        out_specs=[
            pl.BlockSpec((SB, D), lambda sb: (sb, 0)),
            pl.BlockSpec((SB, D), lambda sb: (sb, 0)),
            pl.BlockSpec((SB, E), lambda sb: (sb, 0)),
        ],
        out_shape=[
            jax.ShapeDtypeStruct((S, D), F32),
            jax.ShapeDtypeStruct((S, D), BF),
            jax.ShapeDtypeStruct((S, E), F32),
        ],
    )(aoT, x, w_hi, w_lo, attn_out_b, ln2_w, ln2_b, gate_W, gate_b)


def _gelu_exact(u):
    return u * 0.5 * (1.0 + jax.lax.erf(u * (2.0 ** -0.5)))


def _moe_body(h2_ref, wfull_ref, w1_ref, b1_ref, w2_ref, b2_ref, out_ref):
    e = pl.program_id(1)
    u = jnp.dot(h2_ref[...], w1_ref[0], preferred_element_type=F32)
    u = u + b1_ref[0]
    hid = _gelu_exact(u).astype(BF)
    y = jnp.dot(hid, w2_ref[0], preferred_element_type=F32) + b2_ref[0]
    iota = jax.lax.broadcasted_iota(jnp.int32, (1, E), 1)
    w = jnp.sum(jnp.where(iota == e, wfull_ref[...], 0.0), axis=-1,
                keepdims=True)
    contrib = w * y

    @pl.when(e == 0)
    def _init():
        out_ref[...] = contrib

    @pl.when(e > 0)
    def _acc():
        out_ref[...] += contrib


def _moe_dense(h2b, w_full, exp_W1b, exp_b1, exp_W2b, exp_b2):
    SB = 1024
    return pl.pallas_call(
        _moe_body,
        grid=(S // SB, E),
        in_specs=[
            pl.BlockSpec((SB, D), lambda sb, e: (sb, 0)),
            pl.BlockSpec((SB, E), lambda sb, e: (sb, 0)),
            pl.BlockSpec((1, D, HID), lambda sb, e: (e, 0, 0)),
            pl.BlockSpec((1, 1, HID), lambda sb, e: (e, 0, 0)),
            pl.BlockSpec((1, HID, D), lambda sb, e: (e, 0, 0)),
            pl.BlockSpec((1, 1, D), lambda sb, e: (e, 0, 0)),
        ],
        out_specs=pl.BlockSpec((SB, D), lambda sb, e: (sb, 0)),
        out_shape=jax.ShapeDtypeStruct((S, D), F32),
        compiler_params=pltpu.CompilerParams(
            dimension_semantics=("arbitrary", "arbitrary")),
    )(h2b, w_full, exp_W1b, exp_b1, exp_W2b, exp_b2)


def _final_body(x2_ref, moe_ref, w_ref, b_ref, out_ref):
    z = x2_ref[...] + moe_ref[...]
    m = jnp.sum(z, axis=0, keepdims=True) * (1.0 / S)
    out_ref[...] = jnp.dot(m, w_ref[...], precision=HIGHEST,
                           preferred_element_type=F32) + b_ref[...]


def _final(x2, moe, fc_W, fc_b):
    return pl.pallas_call(
        _final_body,
        grid=(1,),
        in_specs=[
            pl.BlockSpec((S, D), lambda i: (0, 0)),
            pl.BlockSpec((S, D), lambda i: (0, 0)),
            pl.BlockSpec((D, D), lambda i: (0, 0)),
            pl.BlockSpec((D,), lambda i: (0,)),
        ],
        out_specs=pl.BlockSpec((1, D), lambda i: (0, 0)),
        out_shape=jax.ShapeDtypeStruct((1, D), F32),
    )(x2, moe, fc_W, fc_b)


def kernel(x, qkv_W, qkv_b, attn_out_W, attn_out_b, gate_W, gate_b,
           exp_W1, exp_b1, exp_W2, exp_b2, ln1_w, ln1_b, ln2_w, ln2_b,
           fc_W, fc_b):
    xs = x.reshape(S, D)
    qkv_hi, qkv_lo = _ln_qkv(xs, qkv_W, qkv_b, ln1_w, ln1_b)
    aoT = _attention(qkv_hi, qkv_lo)
    x2, h2b, w_full = _proj_router(aoT, xs, attn_out_W, attn_out_b,
                                   ln2_w, ln2_b, gate_W, gate_b)
    moe = _moe_dense(h2b, w_full,
                     exp_W1.astype(BF), exp_b1.reshape(E, 1, HID),
                     exp_W2.astype(BF), exp_b2.reshape(E, 1, D))
    return _final(x2, moe, fc_W, fc_b)
